# baseline (speedup 1.0000x reference)
import functools

import jax
import jax.numpy as jnp
from jax import lax
from jax.experimental import pallas as pl
from jax.experimental.pallas import tpu as pltpu
from jax.experimental.pallas import tpu_sc as plsc

_NC = 2
_NS = 16
_LANES = 16
_CHUNK = 32
_NBUF = 8


@functools.lru_cache(maxsize=None)
def _build(B, L, D, T):
    W = _NC * _NS
    WPB = W // B
    HALF = T // WPB
    NCHUNK = HALF // _CHUNK
    OUTD = D + 4
    ZROW = B * L
    SENT = jnp.int32(0x3FFFFFFF)

    mesh = plsc.VectorSubcoreMesh(
        core_axis_name="c", subcore_axis_name="s",
        num_cores=_NC, num_subcores=_NS)

    @functools.partial(
        pl.kernel,
        out_type=jax.ShapeDtypeStruct((B * T, OUTD), jnp.float32),
        mesh=mesh,
        compiler_params=pltpu.CompilerParams(needs_layout_passes=False),
        scratch_types=[
            pltpu.VMEM((L,), jnp.int32),
            pltpu.VMEM((L + _LANES,), jnp.int32),
            pltpu.VMEM((HALF,), jnp.int32),
            pltpu.VMEM((HALF,), jnp.int32),
            pltpu.VMEM((_NBUF, _CHUNK, D + 4), jnp.float32),
            pltpu.VMEM((HALF * 4,), jnp.float32),
            pltpu.SemaphoreType.DMA((_NBUF,)),
            pltpu.SemaphoreType.DMA((_NBUF,)),
            pltpu.SemaphoreType.DMA,
        ],
    )
    def sc_expand(enc_hbm, dur_hbm, fr_hbm, out_hbm,
                  dur_v, cum_v, m_v, idx_v, gbuf, fbuf, g_sems, w_sems, f_sem):
        wid = lax.axis_index("s") * _NC + lax.axis_index("c")
        b = wid % B
        start_t = (wid // B) * HALF
        out0 = b * T + start_t

        lane = lax.iota(jnp.int32, _LANES)

        fr_src = fr_hbm.at[b, pl.ds(start_t * 4, HALF * 4)]
        pltpu.async_copy(fr_src, fbuf, f_sem)

        pltpu.sync_copy(dur_hbm.at[b], dur_v)

        cum_v[pl.ds(L, _LANES)] = jnp.full((_LANES,), SENT, jnp.int32)

        def cs_body(j, run):
            x = dur_v[pl.ds(j * _LANES, _LANES)]
            s = plsc.cumsum(x) + run
            cum_v[pl.ds(j * _LANES, _LANES)] = s
            return jnp.max(s)

        mel_len = lax.fori_loop(0, L // _LANES, cs_body, jnp.int32(0))

        def z_body(i, _):
            m_v[pl.ds(i * _LANES, _LANES)] = jnp.zeros((_LANES,), jnp.int32)
            return 0

        lax.fori_loop(0, HALF // _LANES, z_body, 0)

        one = jnp.ones((_LANES,), jnp.int32)
        zero = jnp.zeros((_LANES,), jnp.int32)

        def sc_body(j, cnt):
            c16 = cum_v[pl.ds(j * _LANES, _LANES)]
            cnx = cum_v[pl.ds(j * _LANES + 1, _LANES)]
            cnt = cnt + jnp.sum(jnp.where(c16 < start_t, one, zero))
            pos = c16 - start_t
            keep = (c16 != cnx) & (pos >= 0) & (pos < HALF)
            vals = j * _LANES + lane + 1
            plsc.store_scatter(m_v, [pos], vals, mask=keep)
            return cnt

        seed = lax.fori_loop(0, L // _LANES, sc_body, jnp.int32(0))

        def mx_body(i, run):
            v = m_v[pl.ds(i * _LANES, _LANES)]
            s = jnp.maximum(plsc.cummax(v), run)
            t16 = start_t + i * _LANES + lane
            g = jnp.where(t16 < mel_len, b * L + s, jnp.int32(ZROW))
            idx_v[pl.ds(i * _LANES, _LANES)] = g
            return jnp.max(s)

        lax.fori_loop(0, HALF // _LANES, mx_body, seed)

        def g_src(c):
            return enc_hbm.at[idx_v.at[pl.ds(c * _CHUNK, _CHUNK)]]

        def g_dst(s):
            return gbuf.at[s, :, pl.ds(0, D)]

        def w_dst(c):
            return out_hbm.at[pl.ds(out0 + c * _CHUNK, _CHUNK)]

        rpat = lax.shift_right_logical(lane, 2)
        cpat = D + (lane & 3)

        pltpu.make_async_copy(fr_src, fbuf, f_sem).wait()

        for c in range(min(_NBUF, NCHUNK)):
            pltpu.async_copy(g_src(c), g_dst(c), g_sems.at[c])
        for c in range(NCHUNK):
            s = c % _NBUF
            pltpu.make_async_copy(g_src(c), g_dst(s), g_sems.at[s]).wait()

            def mg_body(i2, _):
                vals = fbuf[pl.ds(c * _CHUNK * 4 + i2 * _LANES, _LANES)]
                plsc.store_scatter(gbuf.at[s], [i2 * 4 + rpat, cpat], vals)
                return 0

            lax.fori_loop(0, _CHUNK * 4 // _LANES, mg_body, 0)
            pltpu.async_copy(gbuf.at[s], w_dst(c), w_sems.at[s])
            if c + _NBUF < NCHUNK:
                pltpu.make_async_copy(gbuf.at[s], w_dst(c), w_sems.at[s]).wait()
                pltpu.async_copy(g_src(c + _NBUF), g_dst(s), g_sems.at[s])
        for c in range(max(0, NCHUNK - _NBUF), NCHUNK):
            s = c % _NBUF
            pltpu.make_async_copy(gbuf.at[s], w_dst(c), w_sems.at[s]).wait()

    return sc_expand


def kernel(encoder_outputs, durations, frames_positions, input_lengths):
    B, L, D = encoder_outputs.shape
    T, DP = frames_positions.shape[1], frames_positions.shape[2]
    enc = jnp.pad(encoder_outputs.reshape(B * L, D), ((0, 8), (0, 0)))
    fr = frames_positions.reshape(B, T * DP)
    out = _build(B, L, D, T)(enc, durations, fr)
    return out.reshape(B, T, D + DP)


# speedup vs baseline: 7.2314x; 7.2314x over previous
import functools

import jax
import jax.numpy as jnp
from jax import lax
from jax.experimental import pallas as pl
from jax.experimental.pallas import tpu as pltpu
from jax.experimental.pallas import tpu_sc as plsc

_NC = 2
_NS = 16
_LANES = 16
_CHUNK = 64
_NBUF = 2


@functools.lru_cache(maxsize=None)
def _build(B, L, D, T):
    W = _NC * _NS
    assert W == 2 * B and D == 256 and T % (2 * _CHUNK) == 0
    NPAIR = T // (2 * _CHUNK)
    CW0, CW1 = D // 2, D // 2 + 4
    ZROW = L
    SENT = jnp.int32(0x3FFFFFFF)

    mesh = plsc.VectorSubcoreMesh(
        core_axis_name="c", subcore_axis_name="s",
        num_cores=_NC, num_subcores=_NS)

    @functools.partial(
        pl.kernel,
        out_type=jax.ShapeDtypeStruct((B * T, D + 4), jnp.float32),
        mesh=mesh,
        compiler_params=pltpu.CompilerParams(needs_layout_passes=False),
        scratch_types=[
            pltpu.VMEM((L,), jnp.int32),
            pltpu.VMEM((L + _LANES,), jnp.int32),
            pltpu.VMEM((T,), jnp.int32),
            pltpu.VMEM((L + 8, D // 2), jnp.float32),
            pltpu.VMEM((_NBUF, _CHUNK, D // 2 + 4), jnp.float32),
            pltpu.VMEM((T * 4,), jnp.float32),
            pltpu.SemaphoreType.DMA,
            pltpu.SemaphoreType.DMA((_NBUF,)),
        ],
    )
    def sc_expand(enc_hbm, dur_hbm, fr_hbm, out_hbm,
                  dur_v, cum_v, m_v, stg, gbuf, fbuf, s_sem, w_sems):
        wid = lax.axis_index("s") * _NC + lax.axis_index("c")
        b = wid % B
        ch = wid // B

        lane = lax.iota(jnp.int32, _LANES)
        zv16 = jnp.zeros((_LANES,), jnp.float32)

        pltpu.sync_copy(dur_hbm.at[b], dur_v)

        cum_v[pl.ds(L, _LANES)] = jnp.full((_LANES,), SENT, jnp.int32)

        def cs_body(j, run):
            x = dur_v[pl.ds(j * _LANES, _LANES)]
            s = plsc.cumsum(x) + run
            cum_v[pl.ds(j * _LANES, _LANES)] = s
            return jnp.max(s)

        mel_len = lax.fori_loop(0, L // _LANES, cs_body, jnp.int32(0))

        def z_body(i, _):
            m_v[pl.ds(i * _LANES, _LANES)] = jnp.zeros((_LANES,), jnp.int32)
            return 0

        lax.fori_loop(0, T // _LANES, z_body, 0)

        def sct_body(j, _):
            c16 = cum_v[pl.ds(j * _LANES, _LANES)]
            cnx = cum_v[pl.ds(j * _LANES + 1, _LANES)]
            keep = (c16 != cnx) & (c16 >= 0) & (c16 < T)
            vals = j * _LANES + lane + 1
            plsc.store_scatter(m_v, [c16], vals, mask=keep)
            return 0

        lax.fori_loop(0, L // _LANES, sct_body, 0)

        def mx_body(i, run):
            v = m_v[pl.ds(i * _LANES, _LANES)]
            s = jnp.maximum(plsc.cummax(v), run)
            t16 = i * _LANES + lane
            g = jnp.where(t16 < mel_len, s, jnp.int32(ZROW))
            m_v[pl.ds(i * _LANES, _LANES)] = g
            return jnp.max(s)

        lax.fori_loop(0, T // _LANES, mx_body, jnp.int32(0))

        for k in range(D // 2 // _LANES):
            stg[ZROW, pl.ds(k * _LANES, _LANES)] = zv16

        rpat = lax.shift_right_logical(lane, 2)
        fcol = D // 2 + (lane & 3)

        def expand_rows(c, sl):
            def grp_body(g2, _):
                q16 = m_v[pl.ds(c * _CHUNK + g2 * _LANES, _LANES)]
                for r in range(_LANES):
                    q = q16[r]
                    row = g2 * _LANES + r
                    for k in range(D // 2 // _LANES):
                        gbuf[sl, row, pl.ds(k * _LANES, _LANES)] = (
                            stg[q, pl.ds(k * _LANES, _LANES)])
                return 0

            lax.fori_loop(0, _CHUNK // _LANES, grp_body, 0)

        def merge_frames(c, sl):
            def mg_body(i2, _):
                vals = fbuf[pl.ds(c * _CHUNK * 4 + i2 * _LANES, _LANES)]
                plsc.store_scatter(gbuf.at[sl], [i2 * 4 + rpat, fcol], vals)
                return 0

            lax.fori_loop(0, _CHUNK * 4 // _LANES, mg_body, 0)

        def run_side(co, width, with_frames):
            stg_src = enc_hbm.at[pl.ds(b * L, L), pl.ds(co, D // 2)]
            pltpu.async_copy(stg_src, stg.at[pl.ds(0, L)], s_sem)
            pltpu.make_async_copy(stg_src, stg.at[pl.ds(0, L)], s_sem).wait()
            if with_frames:
                pltpu.async_copy(fr_hbm.at[b], fbuf, s_sem)
                pltpu.make_async_copy(fr_hbm.at[b], fbuf, s_sem).wait()

            def w_dst(c):
                return out_hbm.at[pl.ds(b * T + c * _CHUNK, _CHUNK),
                                  pl.ds(co, width)]

            def w_src(sl):
                return gbuf.at[sl, :, pl.ds(0, width)]

            def pair_body(i, _):
                for sl in range(_NBUF):
                    c = _NBUF * i + sl

                    @pl.when(i > 0)
                    def _():
                        pltpu.make_async_copy(
                            w_src(sl), w_dst(c - _NBUF), w_sems.at[sl]).wait()

                    expand_rows(c, sl)
                    if with_frames:
                        merge_frames(c, sl)
                    pltpu.async_copy(w_src(sl), w_dst(c), w_sems.at[sl])
                return 0

            lax.fori_loop(0, NPAIR, pair_body, 0)
            for sl in range(_NBUF):
                c = _NBUF * (NPAIR - 1) + sl
                pltpu.make_async_copy(
                    w_src(sl), w_dst(c), w_sems.at[sl]).wait()

        @pl.when(ch == 0)
        def _():
            run_side(0, CW0, False)

        @pl.when(ch == 1)
        def _():
            run_side(D // 2, CW1, True)

    return sc_expand


def kernel(encoder_outputs, durations, frames_positions, input_lengths):
    B, L, D = encoder_outputs.shape
    T, DP = frames_positions.shape[1], frames_positions.shape[2]
    enc = encoder_outputs.reshape(B * L, D)
    fr = frames_positions.reshape(B, T * DP)
    out = _build(B, L, D, T)(enc, durations, fr)
    return out.reshape(B, T, D + DP)


# speedup vs baseline: 7.3433x; 1.0155x over previous
import functools

import jax
import jax.numpy as jnp
from jax import lax
from jax.experimental import pallas as pl
from jax.experimental.pallas import tpu as pltpu
from jax.experimental.pallas import tpu_sc as plsc

_NC = 2
_NS = 16
_LANES = 16
_CHUNK = 64
_NBUF = 2


@functools.lru_cache(maxsize=None)
def _build(B, L, D, T):
    W = _NC * _NS
    assert W == 2 * B and D == 256 and T % (2 * _CHUNK) == 0
    NPAIR = T // (2 * _CHUNK)
    CW0, CW1 = D // 2, D // 2 + 4
    ZROW = L
    SENT = jnp.int32(0x3FFFFFFF)

    mesh = plsc.VectorSubcoreMesh(
        core_axis_name="c", subcore_axis_name="s",
        num_cores=_NC, num_subcores=_NS)

    @functools.partial(
        pl.kernel,
        out_type=jax.ShapeDtypeStruct((B * T, D + 4), jnp.float32),
        mesh=mesh,
        compiler_params=pltpu.CompilerParams(needs_layout_passes=False),
        scratch_types=[
            pltpu.VMEM((L,), jnp.int32),
            pltpu.VMEM((L + _LANES,), jnp.int32),
            pltpu.VMEM((T,), jnp.int32),
            pltpu.VMEM((L + 8, D // 2), jnp.float32),
            pltpu.VMEM((_NBUF, _CHUNK, D // 2 + 4), jnp.float32),
            pltpu.VMEM((T * 4,), jnp.float32),
            pltpu.SemaphoreType.DMA,
            pltpu.SemaphoreType.DMA,
            pltpu.SemaphoreType.DMA((_NBUF,)),
        ],
    )
    def sc_expand(enc_hbm, dur_hbm, fr_hbm, out_hbm,
                  dur_v, cum_v, m_v, stg, gbuf, fbuf, s_sem, f_sem, w_sems):
        wid = lax.axis_index("s") * _NC + lax.axis_index("c")
        b = wid % B
        ch = wid // B

        lane = lax.iota(jnp.int32, _LANES)
        zv16 = jnp.zeros((_LANES,), jnp.float32)

        stg_src = enc_hbm.at[pl.ds(b * L, L), pl.ds(ch * (D // 2), D // 2)]
        stg_dst = stg.at[pl.ds(0, L)]
        pltpu.async_copy(stg_src, stg_dst, s_sem)

        @pl.when(ch == 1)
        def _():
            pltpu.async_copy(fr_hbm.at[b], fbuf, f_sem)

        pltpu.sync_copy(dur_hbm.at[b], dur_v)

        cum_v[pl.ds(L, _LANES)] = jnp.full((_LANES,), SENT, jnp.int32)

        def cs_body(j, run):
            x = dur_v[pl.ds(j * _LANES, _LANES)]
            s = plsc.cumsum(x) + run
            cum_v[pl.ds(j * _LANES, _LANES)] = s
            return jnp.max(s)

        mel_len = lax.fori_loop(0, L // _LANES, cs_body, jnp.int32(0))

        def z_body(i, _):
            m_v[pl.ds(i * _LANES, _LANES)] = jnp.zeros((_LANES,), jnp.int32)
            return 0

        lax.fori_loop(0, T // _LANES, z_body, 0)

        def sct_body(j, _):
            c16 = cum_v[pl.ds(j * _LANES, _LANES)]
            cnx = cum_v[pl.ds(j * _LANES + 1, _LANES)]
            keep = (c16 != cnx) & (c16 >= 0) & (c16 < T)
            vals = j * _LANES + lane + 1
            plsc.store_scatter(m_v, [c16], vals, mask=keep)
            return 0

        lax.fori_loop(0, L // _LANES, sct_body, 0)

        def mx_body(i, run):
            v = m_v[pl.ds(i * _LANES, _LANES)]
            s = jnp.maximum(plsc.cummax(v), run)
            t16 = i * _LANES + lane
            g = jnp.where(t16 < mel_len, s, jnp.int32(ZROW))
            m_v[pl.ds(i * _LANES, _LANES)] = g
            return jnp.max(s)

        lax.fori_loop(0, T // _LANES, mx_body, jnp.int32(0))

        for k in range(D // 2 // _LANES):
            stg[ZROW, pl.ds(k * _LANES, _LANES)] = zv16

        rpat = lax.shift_right_logical(lane, 2)
        fcol = D // 2 + (lane & 3)

        def expand_rows(c, sl):
            def grp_body(g2, _):
                q16 = m_v[pl.ds(c * _CHUNK + g2 * _LANES, _LANES)]
                for r in range(_LANES):
                    q = q16[r]
                    row = g2 * _LANES + r
                    for k in range(D // 2 // _LANES):
                        gbuf[sl, row, pl.ds(k * _LANES, _LANES)] = (
                            stg[q, pl.ds(k * _LANES, _LANES)])
                return 0

            lax.fori_loop(0, _CHUNK // _LANES, grp_body, 0)

        def merge_frames(c, sl):
            def mg_body(i2, _):
                vals = fbuf[pl.ds(c * _CHUNK * 4 + i2 * _LANES, _LANES)]
                plsc.store_scatter(gbuf.at[sl], [i2 * 4 + rpat, fcol], vals)
                return 0

            lax.fori_loop(0, _CHUNK * 4 // _LANES, mg_body, 0)

        def run_side(co, width, with_frames):
            pltpu.make_async_copy(stg_src, stg_dst, s_sem).wait()
            if with_frames:
                pltpu.make_async_copy(fr_hbm.at[b], fbuf, f_sem).wait()

            def w_dst(c):
                return out_hbm.at[pl.ds(b * T + c * _CHUNK, _CHUNK),
                                  pl.ds(co, width)]

            def w_src(sl):
                return gbuf.at[sl, :, pl.ds(0, width)]

            def pair_body(i, _):
                for sl in range(_NBUF):
                    c = _NBUF * i + sl

                    @pl.when(i > 0)
                    def _():
                        pltpu.make_async_copy(
                            w_src(sl), w_dst(c - _NBUF), w_sems.at[sl]).wait()

                    expand_rows(c, sl)
                    if with_frames:
                        merge_frames(c, sl)
                    pltpu.async_copy(w_src(sl), w_dst(c), w_sems.at[sl])
                return 0

            lax.fori_loop(0, NPAIR, pair_body, 0)
            for sl in range(_NBUF):
                c = _NBUF * (NPAIR - 1) + sl
                pltpu.make_async_copy(
                    w_src(sl), w_dst(c), w_sems.at[sl]).wait()

        @pl.when(ch == 0)
        def _():
            run_side(0, CW0, False)

        @pl.when(ch == 1)
        def _():
            run_side(D // 2, CW1, True)

    return sc_expand


def kernel(encoder_outputs, durations, frames_positions, input_lengths):
    B, L, D = encoder_outputs.shape
    T, DP = frames_positions.shape[1], frames_positions.shape[2]
    enc = encoder_outputs.reshape(B * L, D)
    fr = frames_positions.reshape(B, T * DP)
    out = _build(B, L, D, T)(enc, durations, fr)
    return out.reshape(B, T, D + DP)


# speedup vs baseline: 8.9680x; 1.2213x over previous
import functools

import jax
import jax.numpy as jnp
from jax import lax
from jax.experimental import pallas as pl
from jax.experimental.pallas import tpu as pltpu
from jax.experimental.pallas import tpu_sc as plsc

_NC = 2
_NS = 16
_LANES = 16
_CHUNK = 64
_NBUF = 2


@functools.lru_cache(maxsize=None)
def _build(B, L, D, T):
    W = _NC * _NS
    assert W == 2 * B and D == 256 and T % (2 * _CHUNK) == 0
    NPAIR = T // (2 * _CHUNK)
    CW0, CW1 = D // 2, D // 2 + 4
    ZROW = L
    SENT = jnp.int32(0x3FFFFFFF)

    mesh = plsc.VectorSubcoreMesh(
        core_axis_name="c", subcore_axis_name="s",
        num_cores=_NC, num_subcores=_NS)

    @functools.partial(
        pl.kernel,
        out_type=jax.ShapeDtypeStruct((B * T, D + 4), jnp.float32),
        mesh=mesh,
        compiler_params=pltpu.CompilerParams(needs_layout_passes=False),
        scratch_types=[
            pltpu.VMEM((L,), jnp.int32),
            pltpu.VMEM((L + _LANES,), jnp.int32),
            pltpu.VMEM((T,), jnp.int32),
            pltpu.VMEM((L + 8, D // 2), jnp.float32),
            pltpu.VMEM((_NBUF, _CHUNK, D // 2 + 4), jnp.float32),
            pltpu.VMEM((T * 4,), jnp.float32),
            pltpu.SemaphoreType.DMA,
            pltpu.SemaphoreType.DMA,
            pltpu.SemaphoreType.DMA((_NBUF,)),
        ],
    )
    def sc_expand(enc_hbm, dur_hbm, fr_hbm, out_hbm,
                  dur_v, cum_v, m_v, stg, gbuf, fbuf, s_sem, f_sem, w_sems):
        wid = lax.axis_index("s") * _NC + lax.axis_index("c")
        b = wid % B
        ch = wid // B

        lane = lax.iota(jnp.int32, _LANES)
        zv16 = jnp.zeros((_LANES,), jnp.float32)

        stg_src = enc_hbm.at[pl.ds(b * L, L), pl.ds(ch * (D // 2), D // 2)]
        stg_dst = stg.at[pl.ds(0, L)]
        pltpu.async_copy(stg_src, stg_dst, s_sem)

        @pl.when(ch == 1)
        def _():
            pltpu.async_copy(fr_hbm.at[b], fbuf, f_sem)

        pltpu.sync_copy(dur_hbm.at[b], dur_v)

        cum_v[pl.ds(L, _LANES)] = jnp.full((_LANES,), SENT, jnp.int32)

        def cs_body(j, run):
            x = dur_v[pl.ds(j * _LANES, _LANES)]
            s = plsc.cumsum(x) + run
            cum_v[pl.ds(j * _LANES, _LANES)] = s
            return jnp.max(s)

        mel_len = lax.fori_loop(0, L // _LANES, cs_body, jnp.int32(0))

        @plsc.parallel_loop(0, T // _LANES, unroll=4)
        def z_body(i):
            m_v[pl.ds(i * _LANES, _LANES)] = jnp.zeros((_LANES,), jnp.int32)

        @plsc.parallel_loop(0, L // _LANES, unroll=2)
        def sct_body(j):
            c16 = cum_v[pl.ds(j * _LANES, _LANES)]
            cnx = cum_v[pl.ds(j * _LANES + 1, _LANES)]
            keep = (c16 != cnx) & (c16 >= 0) & (c16 < T)
            vals = j * _LANES + lane + 1
            plsc.store_scatter(m_v, [c16], vals, mask=keep)

        def mx_body(i, run):
            v = m_v[pl.ds(i * _LANES, _LANES)]
            s = jnp.maximum(plsc.cummax(v), run)
            t16 = i * _LANES + lane
            g = jnp.where(t16 < mel_len, s, jnp.int32(ZROW))
            m_v[pl.ds(i * _LANES, _LANES)] = g
            return jnp.max(s)

        lax.fori_loop(0, T // _LANES, mx_body, jnp.int32(0))

        for k in range(D // 2 // _LANES):
            stg[ZROW, pl.ds(k * _LANES, _LANES)] = zv16

        rpat = lax.shift_right_logical(lane, 2)
        fcol = D // 2 + (lane & 3)

        def expand_rows(c, sl):
            @plsc.parallel_loop(0, _CHUNK // _LANES)
            def grp_body(g2):
                q16 = m_v[pl.ds(c * _CHUNK + g2 * _LANES, _LANES)]
                for r in range(_LANES):
                    q = q16[r]
                    row = g2 * _LANES + r
                    for k in range(D // 2 // _LANES):
                        gbuf[sl, row, pl.ds(k * _LANES, _LANES)] = (
                            stg[q, pl.ds(k * _LANES, _LANES)])

        def merge_frames(c, sl):
            @plsc.parallel_loop(0, _CHUNK * 4 // _LANES, unroll=2)
            def mg_body(i2):
                vals = fbuf[pl.ds(c * _CHUNK * 4 + i2 * _LANES, _LANES)]
                plsc.store_scatter(gbuf.at[sl], [i2 * 4 + rpat, fcol], vals)

        def run_side(co, width, with_frames):
            pltpu.make_async_copy(stg_src, stg_dst, s_sem).wait()
            if with_frames:
                pltpu.make_async_copy(fr_hbm.at[b], fbuf, f_sem).wait()

            def w_dst(c):
                return out_hbm.at[pl.ds(b * T + c * _CHUNK, _CHUNK),
                                  pl.ds(co, width)]

            def w_src(sl):
                return gbuf.at[sl, :, pl.ds(0, width)]

            def pair_body(i, _):
                for sl in range(_NBUF):
                    c = _NBUF * i + sl

                    @pl.when(i > 0)
                    def _():
                        pltpu.make_async_copy(
                            w_src(sl), w_dst(c - _NBUF), w_sems.at[sl]).wait()

                    expand_rows(c, sl)
                    if with_frames:
                        merge_frames(c, sl)
                    pltpu.async_copy(w_src(sl), w_dst(c), w_sems.at[sl])
                return 0

            lax.fori_loop(0, NPAIR, pair_body, 0)
            for sl in range(_NBUF):
                c = _NBUF * (NPAIR - 1) + sl
                pltpu.make_async_copy(
                    w_src(sl), w_dst(c), w_sems.at[sl]).wait()

        @pl.when(ch == 0)
        def _():
            run_side(0, CW0, False)

        @pl.when(ch == 1)
        def _():
            run_side(D // 2, CW1, True)

    return sc_expand


def kernel(encoder_outputs, durations, frames_positions, input_lengths):
    B, L, D = encoder_outputs.shape
    T, DP = frames_positions.shape[1], frames_positions.shape[2]
    enc = encoder_outputs.reshape(B * L, D)
    fr = frames_positions.reshape(B, T * DP)
    out = _build(B, L, D, T)(enc, durations, fr)
    return out.reshape(B, T, D + DP)


# speedup vs baseline: 9.1461x; 1.0199x over previous
import functools

import jax
import jax.numpy as jnp
from jax import lax
from jax.experimental import pallas as pl
from jax.experimental.pallas import tpu as pltpu
from jax.experimental.pallas import tpu_sc as plsc

_NC = 2
_NS = 16
_LANES = 16
_CHUNK = 64
_NBUF = 2


@functools.lru_cache(maxsize=None)
def _build(B, L, D, T):
    W = _NC * _NS
    assert W == 2 * B and D == 256 and T % (2 * _CHUNK) == 0
    NPAIR = T // (2 * _CHUNK)
    CW0, CW1 = D // 2, D // 2 + 4
    ZROW = L
    SENT = jnp.int32(0x3FFFFFFF)

    mesh = plsc.VectorSubcoreMesh(
        core_axis_name="c", subcore_axis_name="s",
        num_cores=_NC, num_subcores=_NS)

    @functools.partial(
        pl.kernel,
        out_type=jax.ShapeDtypeStruct((B * T, D + 4), jnp.float32),
        mesh=mesh,
        compiler_params=pltpu.CompilerParams(needs_layout_passes=False),
        scratch_types=[
            pltpu.VMEM((L,), jnp.int32),
            pltpu.VMEM((L + _LANES,), jnp.int32),
            pltpu.VMEM((T,), jnp.int32),
            pltpu.VMEM((L + 8, D // 2), jnp.float32),
            pltpu.VMEM((_NBUF, _CHUNK, D // 2 + 4), jnp.float32),
            pltpu.VMEM((T * 4,), jnp.float32),
            pltpu.SemaphoreType.DMA,
            pltpu.SemaphoreType.DMA,
            pltpu.SemaphoreType.DMA((_NBUF,)),
        ],
    )
    def sc_expand(enc_hbm, dur_hbm, fr_hbm, out_hbm,
                  dur_v, cum_v, m_v, stg, gbuf, fbuf, s_sem, f_sem, w_sems):
        wid = lax.axis_index("s") * _NC + lax.axis_index("c")
        b = wid % B
        ch = wid // B

        lane = lax.iota(jnp.int32, _LANES)
        zv16 = jnp.zeros((_LANES,), jnp.float32)

        stg_src = enc_hbm.at[pl.ds(b * L, L), pl.ds(ch * (D // 2), D // 2)]
        stg_dst = stg.at[pl.ds(0, L)]
        pltpu.async_copy(stg_src, stg_dst, s_sem)

        @pl.when(ch == 1)
        def _():
            pltpu.async_copy(fr_hbm.at[b], fbuf, f_sem)

        pltpu.sync_copy(dur_hbm.at[b], dur_v)

        cum_v[pl.ds(L, _LANES)] = jnp.full((_LANES,), SENT, jnp.int32)

        def cs_body(j, run):
            x = dur_v[pl.ds(j * _LANES, _LANES)]
            s = plsc.cumsum(x) + run
            cum_v[pl.ds(j * _LANES, _LANES)] = s
            return jnp.max(s)

        mel_len = lax.fori_loop(0, L // _LANES, cs_body, jnp.int32(0))

        @plsc.parallel_loop(0, T // _LANES, unroll=4)
        def z_body(i):
            m_v[pl.ds(i * _LANES, _LANES)] = jnp.zeros((_LANES,), jnp.int32)

        @plsc.parallel_loop(0, L // _LANES, unroll=2)
        def sct_body(j):
            c16 = cum_v[pl.ds(j * _LANES, _LANES)]
            cnx = cum_v[pl.ds(j * _LANES + 1, _LANES)]
            keep = (c16 != cnx) & (c16 >= 0) & (c16 < T)
            vals = j * _LANES + lane + 1
            plsc.store_scatter(m_v, [c16], vals, mask=keep)

        def mx_body(i, run):
            v = m_v[pl.ds(i * _LANES, _LANES)]
            s = jnp.maximum(plsc.cummax(v), run)
            t16 = i * _LANES + lane
            g = jnp.where(t16 < mel_len, s, jnp.int32(ZROW))
            m_v[pl.ds(i * _LANES, _LANES)] = g
            return jnp.max(s)

        lax.fori_loop(0, T // _LANES, mx_body, jnp.int32(0))

        for k in range(D // 2 // _LANES):
            stg[ZROW, pl.ds(k * _LANES, _LANES)] = zv16

        rpat = lax.shift_right_logical(lane, 2)
        fcol = D // 2 + (lane & 3)

        def expand_rows(c, sl):
            @plsc.parallel_loop(0, _CHUNK // _LANES, unroll=2)
            def grp_body(g2):
                q16 = m_v[pl.ds(c * _CHUNK + g2 * _LANES, _LANES)]
                for r in range(_LANES):
                    q = q16[r]
                    row = g2 * _LANES + r
                    for k in range(D // 2 // _LANES):
                        gbuf[sl, row, pl.ds(k * _LANES, _LANES)] = (
                            stg[q, pl.ds(k * _LANES, _LANES)])

        def merge_frames(c, sl):
            @plsc.parallel_loop(0, _CHUNK * 4 // _LANES, unroll=2)
            def mg_body(i2):
                vals = fbuf[pl.ds(c * _CHUNK * 4 + i2 * _LANES, _LANES)]
                plsc.store_scatter(gbuf.at[sl], [i2 * 4 + rpat, fcol], vals)

        def run_side(co, width, with_frames):
            pltpu.make_async_copy(stg_src, stg_dst, s_sem).wait()
            if with_frames:
                pltpu.make_async_copy(fr_hbm.at[b], fbuf, f_sem).wait()

            def w_dst(c):
                return out_hbm.at[pl.ds(b * T + c * _CHUNK, _CHUNK),
                                  pl.ds(co, width)]

            def w_src(sl):
                return gbuf.at[sl, :, pl.ds(0, width)]

            def pair_body(i, _):
                for sl in range(_NBUF):
                    c = _NBUF * i + sl

                    @pl.when(i > 0)
                    def _():
                        pltpu.make_async_copy(
                            w_src(sl), w_dst(c - _NBUF), w_sems.at[sl]).wait()

                    expand_rows(c, sl)
                    if with_frames:
                        merge_frames(c, sl)
                    pltpu.async_copy(w_src(sl), w_dst(c), w_sems.at[sl])
                return 0

            lax.fori_loop(0, NPAIR, pair_body, 0)
            for sl in range(_NBUF):
                c = _NBUF * (NPAIR - 1) + sl
                pltpu.make_async_copy(
                    w_src(sl), w_dst(c), w_sems.at[sl]).wait()

        @pl.when(ch == 0)
        def _():
            run_side(0, CW0, False)

        @pl.when(ch == 1)
        def _():
            run_side(D // 2, CW1, True)

    return sc_expand


def kernel(encoder_outputs, durations, frames_positions, input_lengths):
    B, L, D = encoder_outputs.shape
    T, DP = frames_positions.shape[1], frames_positions.shape[2]
    enc = encoder_outputs.reshape(B * L, D)
    fr = frames_positions.reshape(B, T * DP)
    out = _build(B, L, D, T)(enc, durations, fr)
    return out.reshape(B, T, D + DP)
